# baseline (speedup 1.0000x reference)
import jax
import jax.numpy as jnp
from jax import lax
from jax.experimental import pallas as pl
from jax.experimental.pallas import tpu as pltpu
from jax.experimental.pallas import tpu_sc as plsc

N_NODES = 10000
D_FEAT = 128
N_EDGES = 320000
M_ITEMS = 1000
EMB = 64
B = 4096

NC = 2
NS = 16
NW = NC * NS
CH = 128
EW = N_EDGES // NW
NCH = 80
E_PAD = NW * NCH * CH
N_ACC = N_NODES + 112
ROWS_PER_TILE = N_ACC // NS


def _f32(x):
    return x.astype(jnp.float32)


def _make_edge_agg(with_cnt: bool):
    mesh = plsc.VectorSubcoreMesh(core_axis_name="c", subcore_axis_name="s")
    out_type = [jax.ShapeDtypeStruct((NC, N_ACC, EMB), jnp.float32)]
    scratch = [
        pltpu.VMEM((NCH, CH), jnp.int32),
        pltpu.VMEM((NCH, CH), jnp.int32),
        pltpu.VMEM_SHARED((N_ACC, EMB), jnp.float32),
        pltpu.VMEM((CH, EMB), jnp.float32),
        pltpu.VMEM((CH, EMB), jnp.float32),
        pltpu.SemaphoreType.DMA,
        pltpu.SemaphoreType.DMA,
        pltpu.SemaphoreType.DMA,
    ]
    if with_cnt:
        out_type.append(jax.ShapeDtypeStruct((NC, N_ACC, 16), jnp.float32))
        scratch += [
            pltpu.VMEM((CH, 16), jnp.float32),
            pltpu.VMEM_SHARED((N_ACC, 16), jnp.float32),
        ]

    def body(p_hbm, src_hbm, dst_hbm, z64_hbm, z16_hbm, ones_hbm, *refs):
        if with_cnt:
            out_hbm, cnt_hbm = refs[0], refs[1]
            refs = refs[2:]
        else:
            out_hbm = refs[0]
            refs = refs[1:]
        (src_v, dst_v, acc_s, rows_a, rows_b, gsem_a, gsem_b,
         csem) = refs[:8]
        if with_cnt:
            ones_v, cnt_s = refs[8], refs[9]
        c = lax.axis_index("c")
        s = lax.axis_index("s")
        wid = s * NC + c
        r0 = s * ROWS_PER_TILE
        pltpu.sync_copy(z64_hbm.at[pl.ds(r0, ROWS_PER_TILE)],
                        acc_s.at[pl.ds(r0, ROWS_PER_TILE)])
        if with_cnt:
            pltpu.sync_copy(z16_hbm.at[pl.ds(r0, ROWS_PER_TILE)],
                            cnt_s.at[pl.ds(r0, ROWS_PER_TILE)])
            pltpu.sync_copy(ones_hbm, ones_v)
        pltpu.sync_copy(src_hbm.at[wid], src_v)
        pltpu.sync_copy(dst_hbm.at[wid], dst_v)
        plsc.subcore_barrier()

        def gather(j, rows, sem):
            pltpu.async_copy(p_hbm.at[src_v.at[j]], rows, sem)

        def gwait(j, rows, sem):
            pltpu.make_async_copy(p_hbm.at[src_v.at[j]], rows, sem).wait()

        def scatter(j, rows):
            pltpu.sync_copy(rows, acc_s.at[dst_v.at[j]], add=True)
            if with_cnt:
                pltpu.sync_copy(ones_v, cnt_s.at[dst_v.at[j]], add=True)

        gather(0, rows_a, gsem_a)

        def step(k, carry):
            ja = 2 * k
            jb = 2 * k + 1
            gather(jb, rows_b, gsem_b)
            gwait(ja, rows_a, gsem_a)
            scatter(ja, rows_a)

            @pl.when(ja + 2 < NCH)
            def _():
                gather(ja + 2, rows_a, gsem_a)

            gwait(jb, rows_b, gsem_b)
            scatter(jb, rows_b)
            return carry

        lax.fori_loop(0, NCH // 2, step, 0)
        plsc.subcore_barrier()
        pltpu.sync_copy(acc_s.at[pl.ds(r0, ROWS_PER_TILE)],
                        out_hbm.at[c, pl.ds(r0, ROWS_PER_TILE)])
        if with_cnt:
            pltpu.sync_copy(cnt_s.at[pl.ds(r0, ROWS_PER_TILE)],
                            cnt_hbm.at[c, pl.ds(r0, ROWS_PER_TILE)])

    return pl.kernel(body, out_type=tuple(out_type), mesh=mesh,
                     scratch_types=scratch,
                     compiler_params=pltpu.CompilerParams(
                         use_tc_tiling_on_sc=False))


_edge_agg_cnt = _make_edge_agg(True)
_edge_agg = _make_edge_agg(False)


_BG = B // NW


def _batch_gather_body(tab_hbm, x_hbm, out_hbm, idx_v, rows_v, sem):
    c = lax.axis_index("c")
    s = lax.axis_index("s")
    base = (s * NC + c) * _BG
    pltpu.sync_copy(x_hbm.at[pl.ds(base, _BG)], idx_v)
    pltpu.async_copy(tab_hbm.at[idx_v], rows_v, sem).wait()
    pltpu.sync_copy(rows_v, out_hbm.at[pl.ds(base, _BG)])


_batch_gather = pl.kernel(
    _batch_gather_body,
    out_type=jax.ShapeDtypeStruct((B, EMB), jnp.float32),
    mesh=plsc.VectorSubcoreMesh(core_axis_name="c", subcore_axis_name="s"),
    scratch_types=[
        pltpu.VMEM((_BG,), jnp.int32),
        pltpu.VMEM((_BG, EMB), jnp.float32),
        pltpu.SemaphoreType.DMA,
    ],
    compiler_params=pltpu.CompilerParams(use_tc_tiling_on_sc=False),
)


_RB = 1000


def _pre_body(nx_ref, rat_ref, w1lT_ref, w1rT_ref, wencT_ref,
              p1_ref, xr_ref, renc_ref):
    nx = nx_ref[...]
    p1_ref[...] = jnp.dot(nx, w1lT_ref[...],
                          preferred_element_type=jnp.float32)
    xr_ref[...] = jnp.dot(nx, w1rT_ref[...],
                          preferred_element_type=jnp.float32)
    renc_ref[...] = jnp.dot(rat_ref[...], wencT_ref[...],
                            preferred_element_type=jnp.float32)


def _tc_pre(node_x, rating_mat, w1lT, w1rT, wencT):
    n = node_x.shape[0]
    grid = (n // _RB,)
    return pl.pallas_call(
        _pre_body,
        grid=grid,
        in_specs=[
            pl.BlockSpec((_RB, D_FEAT), lambda i: (i, 0)),
            pl.BlockSpec((_RB, M_ITEMS), lambda i: (i, 0)),
            pl.BlockSpec((D_FEAT, EMB), lambda i: (0, 0)),
            pl.BlockSpec((D_FEAT, EMB), lambda i: (0, 0)),
            pl.BlockSpec((M_ITEMS, EMB), lambda i: (0, 0)),
        ],
        out_specs=[
            pl.BlockSpec((_RB, EMB), lambda i: (i, 0)),
            pl.BlockSpec((_RB, EMB), lambda i: (i, 0)),
            pl.BlockSpec((_RB, EMB), lambda i: (i, 0)),
        ],
        out_shape=[
            jax.ShapeDtypeStruct((n, EMB), jnp.float32),
            jax.ShapeDtypeStruct((n, EMB), jnp.float32),
            jax.ShapeDtypeStruct((n, EMB), jnp.float32),
        ],
    )(node_x, rating_mat, w1lT, w1rT, wencT)


def _mid_body(agg_ref, cnt_ref, xr_ref, mask2_ref, b1l_ref,
              w2lT_ref, w2rT_ref, p2_ref, hr_ref):
    a = agg_ref[0] + agg_ref[1]
    cnt = cnt_ref[0, :, 0:1] + cnt_ref[1, :, 0:1]
    mean = a / jnp.maximum(cnt, 1.0)
    h = jnp.maximum(mean + b1l_ref[...] + xr_ref[...], 0.0) * mask2_ref[...]
    p2_ref[...] = jnp.dot(h, w2lT_ref[...],
                          preferred_element_type=jnp.float32)
    hr_ref[...] = jnp.dot(h, w2rT_ref[...],
                          preferred_element_type=jnp.float32)


def _tc_mid(agg1, cnt, xr, mask2, b1l, w2lT, w2rT):
    n = xr.shape[0]
    grid = (n // _RB,)
    return pl.pallas_call(
        _mid_body,
        grid=grid,
        in_specs=[
            pl.BlockSpec((NC, _RB, EMB), lambda i: (0, i, 0)),
            pl.BlockSpec((NC, _RB, 16), lambda i: (0, i, 0)),
            pl.BlockSpec((_RB, EMB), lambda i: (i, 0)),
            pl.BlockSpec((_RB, EMB), lambda i: (i, 0)),
            pl.BlockSpec((1, EMB), lambda i: (0, 0)),
            pl.BlockSpec((EMB, EMB), lambda i: (0, 0)),
            pl.BlockSpec((EMB, EMB), lambda i: (0, 0)),
        ],
        out_specs=[
            pl.BlockSpec((_RB, EMB), lambda i: (i, 0)),
            pl.BlockSpec((_RB, EMB), lambda i: (i, 0)),
        ],
        out_shape=[
            jax.ShapeDtypeStruct((n, EMB), jnp.float32),
            jax.ShapeDtypeStruct((n, EMB), jnp.float32),
        ],
    )(agg1, cnt, xr, mask2, b1l, w2lT, w2rT)


def _post_body(agg_ref, cnt_ref, hr_ref, renc_ref, bias_ref, s_ref):
    a = agg_ref[0] + agg_ref[1]
    cnt = cnt_ref[0, :, 0:1] + cnt_ref[1, :, 0:1]
    mean = a / jnp.maximum(cnt, 1.0)
    s_ref[...] = mean + hr_ref[...] + renc_ref[...] + bias_ref[...]


def _tc_post(agg2, cnt, hr, renc, bias):
    n = hr.shape[0]
    grid = (n // _RB,)
    return pl.pallas_call(
        _post_body,
        grid=grid,
        in_specs=[
            pl.BlockSpec((NC, _RB, EMB), lambda i: (0, i, 0)),
            pl.BlockSpec((NC, _RB, 16), lambda i: (0, i, 0)),
            pl.BlockSpec((_RB, EMB), lambda i: (i, 0)),
            pl.BlockSpec((_RB, EMB), lambda i: (i, 0)),
            pl.BlockSpec((1, EMB), lambda i: (0, 0)),
        ],
        out_specs=pl.BlockSpec((_RB, EMB), lambda i: (i, 0)),
        out_shape=jax.ShapeDtypeStruct((n, EMB), jnp.float32),
    )(agg2, cnt, hr, renc, bias)


_DB = 512


def _dec_body(sx_ref, wdecT_ref, bdec_ref, out_ref):
    t = jax.nn.sigmoid(sx_ref[...])
    y = jnp.dot(t, wdecT_ref[...], preferred_element_type=jnp.float32)
    out_ref[...] = jax.nn.sigmoid(y + bdec_ref[...])


def _tc_dec(sx, wdecT, bdec):
    grid = (B // _DB,)
    return pl.pallas_call(
        _dec_body,
        grid=grid,
        in_specs=[
            pl.BlockSpec((_DB, EMB), lambda i: (i, 0)),
            pl.BlockSpec((EMB, M_ITEMS), lambda i: (0, 0)),
            pl.BlockSpec((1, M_ITEMS), lambda i: (0, 0)),
        ],
        out_specs=pl.BlockSpec((_DB, M_ITEMS), lambda i: (i, 0)),
        out_shape=jax.ShapeDtypeStruct((B, M_ITEMS), jnp.float32),
    )(sx, wdecT, bdec)


def kernel(x, rating_mat, node_x, edge_index, user_table,
           w1l, b1l, w1r, w2l, b2l, w2r,
           w_enc, b_enc, w_dec, b_dec):
    del user_table
    x = x.astype(jnp.int32)
    src = edge_index[0].astype(jnp.int32)
    dst = edge_index[1].astype(jnp.int32)
    pad = E_PAD - N_EDGES
    src_p = jnp.concatenate([src, jnp.zeros((pad,), jnp.int32)])
    dst_p = jnp.concatenate([dst, jnp.full((pad,), N_NODES, jnp.int32)])
    src_p = src_p.reshape(NW, NCH, CH)
    dst_p = dst_p.reshape(NW, NCH, CH)
    z64 = jnp.zeros((N_ACC, EMB), jnp.float32)
    z16 = jnp.zeros((N_ACC, 16), jnp.float32)
    ones = jnp.ones((CH, 16), jnp.float32)

    keep = jax.random.bernoulli(jax.random.key(42), 0.5, (N_NODES, EMB))
    mask2 = keep.astype(jnp.float32) * 2.0

    p1, xr, renc = _tc_pre(node_x, rating_mat, _f32(w1l.T), _f32(w1r.T),
                           _f32(w_enc.T))
    agg1, cnt = _edge_agg_cnt(p1, src_p, dst_p, z64, z16, ones)
    p2, hr = _tc_mid(agg1, cnt, xr, mask2, b1l.reshape(1, EMB),
                     _f32(w2l.T), _f32(w2r.T))
    (agg2,) = _edge_agg(p2, src_p, dst_p, z64, z16, ones)
    bias = (b2l + b_enc).reshape(1, EMB)
    s_tab = _tc_post(agg2, cnt, hr, renc, bias)
    sx = _batch_gather(s_tab, x)
    return _tc_dec(sx, _f32(w_dec.T), b_dec.reshape(1, M_ITEMS))


# speedup vs baseline: 1.3708x; 1.3708x over previous
import jax
import jax.numpy as jnp
from jax import lax
from jax.experimental import pallas as pl
from jax.experimental.pallas import tpu as pltpu
from jax.experimental.pallas import tpu_sc as plsc

N_NODES = 10000
D_FEAT = 128
N_EDGES = 320000
M_ITEMS = 1000
EMB = 64
B = 4096

NC = 2
NS = 16
NW = NC * NS
CH = 128
EW = N_EDGES // NW
NCH = 79
E_PAD = NW * NCH * CH
N_ACC = N_NODES + 112
ROWS_PER_TILE = N_ACC // NS


def _f32(x):
    return x.astype(jnp.float32)


def _make_edge_agg(with_cnt: bool):
    mesh = plsc.VectorSubcoreMesh(core_axis_name="c", subcore_axis_name="s")
    out_type = [jax.ShapeDtypeStruct((NC, N_ACC, EMB), jnp.float32)]
    scratch = [
        pltpu.VMEM((NCH, CH), jnp.int32),
        pltpu.VMEM((NCH, CH), jnp.int32),
        pltpu.VMEM((CH, EMB), jnp.float32),
        pltpu.VMEM((CH, EMB), jnp.float32),
        pltpu.VMEM_SHARED((N_ACC, EMB), jnp.float32),
        pltpu.SemaphoreType.DMA,
        pltpu.SemaphoreType.DMA,
    ]
    if with_cnt:
        out_type.append(jax.ShapeDtypeStruct((NC, N_ACC, 16), jnp.float32))
        scratch += [
            pltpu.VMEM((CH, 16), jnp.float32),
            pltpu.VMEM_SHARED((N_ACC, 16), jnp.float32),
        ]

    def body(p_hbm, src_hbm, dst_hbm, z64_hbm, z16_hbm, ones_hbm, *refs):
        if with_cnt:
            out_hbm, cnt_hbm = refs[0], refs[1]
            refs = refs[2:]
        else:
            out_hbm = refs[0]
            refs = refs[1:]
        (src_v, dst_v, rows_a, rows_b, acc_s, gsem_a, gsem_b) = refs[:7]
        if with_cnt:
            ones_v, cnt_s = refs[7], refs[8]
        c = lax.axis_index("c")
        s = lax.axis_index("s")
        wid = s * NC + c
        r0 = s * ROWS_PER_TILE
        pltpu.sync_copy(z64_hbm.at[pl.ds(r0, ROWS_PER_TILE)],
                        acc_s.at[pl.ds(r0, ROWS_PER_TILE)])
        if with_cnt:
            pltpu.sync_copy(z16_hbm.at[pl.ds(r0, ROWS_PER_TILE)],
                            cnt_s.at[pl.ds(r0, ROWS_PER_TILE)])
            pltpu.sync_copy(ones_hbm, ones_v)
        pltpu.sync_copy(src_hbm.at[wid], src_v)
        pltpu.sync_copy(dst_hbm.at[wid], dst_v)
        plsc.subcore_barrier()

        def gather(j, rows, sem):
            pltpu.async_copy(p_hbm.at[src_v.at[j]], rows, sem)

        def gwait(j, rows, sem):
            pltpu.make_async_copy(p_hbm.at[src_v.at[j]], rows, sem).wait()

        def scatter(j, rows):
            pltpu.sync_copy(rows, acc_s.at[dst_v.at[j]], add=True)
            if with_cnt:
                pltpu.sync_copy(ones_v, cnt_s.at[dst_v.at[j]], add=True)

        gather(0, rows_a, gsem_a)

        def step(k, carry):
            ja = 2 * k
            jb = 2 * k + 1

            @pl.when(jb < NCH)
            def _():
                gather(jb, rows_b, gsem_b)

            gwait(ja, rows_a, gsem_a)
            scatter(ja, rows_a)

            @pl.when(ja + 2 < NCH)
            def _():
                gather(ja + 2, rows_a, gsem_a)

            @pl.when(jb < NCH)
            def _():
                gwait(jb, rows_b, gsem_b)
                scatter(jb, rows_b)

            return carry

        lax.fori_loop(0, (NCH + 1) // 2, step, 0)
        plsc.subcore_barrier()
        pltpu.sync_copy(acc_s.at[pl.ds(r0, ROWS_PER_TILE)],
                        out_hbm.at[c, pl.ds(r0, ROWS_PER_TILE)])
        if with_cnt:
            pltpu.sync_copy(cnt_s.at[pl.ds(r0, ROWS_PER_TILE)],
                            cnt_hbm.at[c, pl.ds(r0, ROWS_PER_TILE)])

    return pl.kernel(body, out_type=tuple(out_type), mesh=mesh,
                     scratch_types=scratch,
                     compiler_params=pltpu.CompilerParams(
                         use_tc_tiling_on_sc=False))


_edge_agg_cnt = _make_edge_agg(True)
_edge_agg = _make_edge_agg(False)


_BG = B // NW


def _batch_gather_body(tab_hbm, x_hbm, out_hbm, idx_v, rows_v, sem):
    c = lax.axis_index("c")
    s = lax.axis_index("s")
    base = (s * NC + c) * _BG
    pltpu.sync_copy(x_hbm.at[pl.ds(base, _BG)], idx_v)
    pltpu.async_copy(tab_hbm.at[idx_v], rows_v, sem).wait()
    pltpu.sync_copy(rows_v, out_hbm.at[pl.ds(base, _BG)])


_batch_gather = pl.kernel(
    _batch_gather_body,
    out_type=jax.ShapeDtypeStruct((B, EMB), jnp.float32),
    mesh=plsc.VectorSubcoreMesh(core_axis_name="c", subcore_axis_name="s"),
    scratch_types=[
        pltpu.VMEM((_BG,), jnp.int32),
        pltpu.VMEM((_BG, EMB), jnp.float32),
        pltpu.SemaphoreType.DMA,
    ],
    compiler_params=pltpu.CompilerParams(use_tc_tiling_on_sc=False),
)


_RB = 1000


def _pre_body(nx_ref, rat_ref, w1lT_ref, w1rT_ref, wencT_ref,
              p1_ref, xr_ref, renc_ref):
    nx = nx_ref[...]
    p1_ref[...] = jnp.dot(nx, w1lT_ref[...],
                          preferred_element_type=jnp.float32)
    xr_ref[...] = jnp.dot(nx, w1rT_ref[...],
                          preferred_element_type=jnp.float32)
    renc_ref[...] = jnp.dot(rat_ref[...], wencT_ref[...],
                            preferred_element_type=jnp.float32)


def _tc_pre(node_x, rating_mat, w1lT, w1rT, wencT):
    n = node_x.shape[0]
    grid = (n // _RB,)
    return pl.pallas_call(
        _pre_body,
        grid=grid,
        in_specs=[
            pl.BlockSpec((_RB, D_FEAT), lambda i: (i, 0)),
            pl.BlockSpec((_RB, M_ITEMS), lambda i: (i, 0)),
            pl.BlockSpec((D_FEAT, EMB), lambda i: (0, 0)),
            pl.BlockSpec((D_FEAT, EMB), lambda i: (0, 0)),
            pl.BlockSpec((M_ITEMS, EMB), lambda i: (0, 0)),
        ],
        out_specs=[
            pl.BlockSpec((_RB, EMB), lambda i: (i, 0)),
            pl.BlockSpec((_RB, EMB), lambda i: (i, 0)),
            pl.BlockSpec((_RB, EMB), lambda i: (i, 0)),
        ],
        out_shape=[
            jax.ShapeDtypeStruct((n, EMB), jnp.float32),
            jax.ShapeDtypeStruct((n, EMB), jnp.float32),
            jax.ShapeDtypeStruct((n, EMB), jnp.float32),
        ],
    )(node_x, rating_mat, w1lT, w1rT, wencT)


def _mid_body(agg_ref, cnt_ref, xr_ref, mask2_ref, b1l_ref,
              w2lT_ref, w2rT_ref, p2_ref, hr_ref):
    a = agg_ref[0] + agg_ref[1]
    cnt = cnt_ref[0, :, 0:1] + cnt_ref[1, :, 0:1]
    mean = a / jnp.maximum(cnt, 1.0)
    h = jnp.maximum(mean + b1l_ref[...] + xr_ref[...], 0.0) * mask2_ref[...]
    p2_ref[...] = jnp.dot(h, w2lT_ref[...],
                          preferred_element_type=jnp.float32)
    hr_ref[...] = jnp.dot(h, w2rT_ref[...],
                          preferred_element_type=jnp.float32)


def _tc_mid(agg1, cnt, xr, mask2, b1l, w2lT, w2rT):
    n = xr.shape[0]
    grid = (n // _RB,)
    return pl.pallas_call(
        _mid_body,
        grid=grid,
        in_specs=[
            pl.BlockSpec((NC, _RB, EMB), lambda i: (0, i, 0)),
            pl.BlockSpec((NC, _RB, 16), lambda i: (0, i, 0)),
            pl.BlockSpec((_RB, EMB), lambda i: (i, 0)),
            pl.BlockSpec((_RB, EMB), lambda i: (i, 0)),
            pl.BlockSpec((1, EMB), lambda i: (0, 0)),
            pl.BlockSpec((EMB, EMB), lambda i: (0, 0)),
            pl.BlockSpec((EMB, EMB), lambda i: (0, 0)),
        ],
        out_specs=[
            pl.BlockSpec((_RB, EMB), lambda i: (i, 0)),
            pl.BlockSpec((_RB, EMB), lambda i: (i, 0)),
        ],
        out_shape=[
            jax.ShapeDtypeStruct((n, EMB), jnp.float32),
            jax.ShapeDtypeStruct((n, EMB), jnp.float32),
        ],
    )(agg1, cnt, xr, mask2, b1l, w2lT, w2rT)


def _post_body(agg_ref, cnt_ref, hr_ref, renc_ref, bias_ref, s_ref):
    a = agg_ref[0] + agg_ref[1]
    cnt = cnt_ref[0, :, 0:1] + cnt_ref[1, :, 0:1]
    mean = a / jnp.maximum(cnt, 1.0)
    s_ref[...] = mean + hr_ref[...] + renc_ref[...] + bias_ref[...]


def _tc_post(agg2, cnt, hr, renc, bias):
    n = hr.shape[0]
    grid = (n // _RB,)
    return pl.pallas_call(
        _post_body,
        grid=grid,
        in_specs=[
            pl.BlockSpec((NC, _RB, EMB), lambda i: (0, i, 0)),
            pl.BlockSpec((NC, _RB, 16), lambda i: (0, i, 0)),
            pl.BlockSpec((_RB, EMB), lambda i: (i, 0)),
            pl.BlockSpec((_RB, EMB), lambda i: (i, 0)),
            pl.BlockSpec((1, EMB), lambda i: (0, 0)),
        ],
        out_specs=pl.BlockSpec((_RB, EMB), lambda i: (i, 0)),
        out_shape=jax.ShapeDtypeStruct((n, EMB), jnp.float32),
    )(agg2, cnt, hr, renc, bias)


_DB = 512


def _dec_body(sx_ref, wdecT_ref, bdec_ref, out_ref):
    t = jax.nn.sigmoid(sx_ref[...])
    y = jnp.dot(t, wdecT_ref[...], preferred_element_type=jnp.float32)
    out_ref[...] = jax.nn.sigmoid(y + bdec_ref[...])


def _tc_dec(sx, wdecT, bdec):
    grid = (B // _DB,)
    return pl.pallas_call(
        _dec_body,
        grid=grid,
        in_specs=[
            pl.BlockSpec((_DB, EMB), lambda i: (i, 0)),
            pl.BlockSpec((EMB, M_ITEMS), lambda i: (0, 0)),
            pl.BlockSpec((1, M_ITEMS), lambda i: (0, 0)),
        ],
        out_specs=pl.BlockSpec((_DB, M_ITEMS), lambda i: (i, 0)),
        out_shape=jax.ShapeDtypeStruct((B, M_ITEMS), jnp.float32),
    )(sx, wdecT, bdec)


def kernel(x, rating_mat, node_x, edge_index, user_table,
           w1l, b1l, w1r, w2l, b2l, w2r,
           w_enc, b_enc, w_dec, b_dec):
    del user_table
    x = x.astype(jnp.int32)
    src = edge_index[0].astype(jnp.int32)
    dst = edge_index[1].astype(jnp.int32)
    pad = E_PAD - N_EDGES
    src_p = jnp.concatenate([src, jnp.zeros((pad,), jnp.int32)])
    dst_p = jnp.concatenate([dst, jnp.full((pad,), N_NODES, jnp.int32)])
    src_p = src_p.reshape(NW, NCH, CH)
    dst_p = dst_p.reshape(NW, NCH, CH)
    z64 = jnp.zeros((N_ACC, EMB), jnp.float32)
    z16 = jnp.zeros((N_ACC, 16), jnp.float32)
    ones = jnp.ones((CH, 16), jnp.float32)

    keep = jax.random.bernoulli(jax.random.key(42), 0.5, (N_NODES, EMB))
    mask2 = keep.astype(jnp.float32) * 2.0

    p1, xr, renc = _tc_pre(node_x, rating_mat, _f32(w1l.T), _f32(w1r.T),
                           _f32(w_enc.T))
    agg1, cnt = _edge_agg_cnt(p1, src_p, dst_p, z64, z16, ones)
    p2, hr = _tc_mid(agg1, cnt, xr, mask2, b1l.reshape(1, EMB),
                     _f32(w2l.T), _f32(w2r.T))
    (agg2,) = _edge_agg(p2, src_p, dst_p, z64, z16, ones)
    bias = (b2l + b_enc).reshape(1, EMB)
    s_tab = _tc_post(agg2, cnt, hr, renc, bias)
    sx = _batch_gather(s_tab, x)
    return _tc_dec(sx, _f32(w_dec.T), b_dec.reshape(1, M_ITEMS))


# speedup vs baseline: 1.3721x; 1.0009x over previous
import jax
import jax.numpy as jnp
from jax import lax
from jax.experimental import pallas as pl
from jax.experimental.pallas import tpu as pltpu
from jax.experimental.pallas import tpu_sc as plsc

N_NODES = 10000
D_FEAT = 128
N_EDGES = 320000
M_ITEMS = 1000
EMB = 64
B = 4096

NC = 2
NS = 16
NW = NC * NS
CH = 128
EW = N_EDGES // NW
NCH = 79
E_PAD = NW * NCH * CH
N_ACC = N_NODES + 112
ROWS_PER_TILE = N_ACC // NS


def _f32(x):
    return x.astype(jnp.float32)


def _make_edge_agg(with_cnt: bool):
    mesh = plsc.VectorSubcoreMesh(core_axis_name="c", subcore_axis_name="s")
    out_type = [jax.ShapeDtypeStruct((NC, N_ACC, EMB), jnp.float32)]
    scratch = [
        pltpu.VMEM((NCH, CH), jnp.int32),
        pltpu.VMEM((NCH, CH), jnp.int32),
        pltpu.VMEM((CH, EMB), jnp.float32),
        pltpu.VMEM((CH, EMB), jnp.float32),
        pltpu.VMEM_SHARED((N_ACC, EMB), jnp.float32),
        pltpu.SemaphoreType.DMA,
        pltpu.SemaphoreType.DMA,
    ]
    if with_cnt:
        out_type.append(jax.ShapeDtypeStruct((NC, N_ACC, 16), jnp.float32))
        scratch += [
            pltpu.VMEM((CH, 16), jnp.float32),
            pltpu.VMEM_SHARED((N_ACC, 16), jnp.float32),
        ]

    def body(p_hbm, src_hbm, dst_hbm, z64_hbm, z16_hbm, ones_hbm, *refs):
        if with_cnt:
            out_hbm, cnt_hbm = refs[0], refs[1]
            refs = refs[2:]
        else:
            out_hbm = refs[0]
            refs = refs[1:]
        (src_v, dst_v, rows_a, rows_b, acc_s, gsem_a, gsem_b) = refs[:7]
        if with_cnt:
            ones_v, cnt_s = refs[7], refs[8]
        c = lax.axis_index("c")
        s = lax.axis_index("s")
        wid = s * NC + c
        r0 = s * ROWS_PER_TILE
        pltpu.sync_copy(z64_hbm.at[pl.ds(r0, ROWS_PER_TILE)],
                        acc_s.at[pl.ds(r0, ROWS_PER_TILE)])
        if with_cnt:
            pltpu.sync_copy(z16_hbm.at[pl.ds(r0, ROWS_PER_TILE)],
                            cnt_s.at[pl.ds(r0, ROWS_PER_TILE)])
            pltpu.sync_copy(ones_hbm, ones_v)
        pltpu.sync_copy(src_hbm.at[wid], src_v)
        pltpu.sync_copy(dst_hbm.at[wid], dst_v)
        plsc.subcore_barrier()

        def gather(j, rows, sem):
            pltpu.async_copy(p_hbm.at[src_v.at[j]], rows, sem)

        def gwait(j, rows, sem):
            pltpu.make_async_copy(p_hbm.at[src_v.at[j]], rows, sem).wait()

        def scatter(j, rows):
            pltpu.sync_copy(rows, acc_s.at[dst_v.at[j]], add=True)
            if with_cnt:
                pltpu.sync_copy(ones_v, cnt_s.at[dst_v.at[j]], add=True)

        gather(0, rows_a, gsem_a)

        def step(k, carry):
            ja = 2 * k
            jb = 2 * k + 1

            @pl.when(jb < NCH)
            def _():
                gather(jb, rows_b, gsem_b)

            gwait(ja, rows_a, gsem_a)
            scatter(ja, rows_a)

            @pl.when(ja + 2 < NCH)
            def _():
                gather(ja + 2, rows_a, gsem_a)

            @pl.when(jb < NCH)
            def _():
                gwait(jb, rows_b, gsem_b)
                scatter(jb, rows_b)

            return carry

        lax.fori_loop(0, (NCH + 1) // 2, step, 0)
        plsc.subcore_barrier()
        pltpu.sync_copy(acc_s.at[pl.ds(r0, ROWS_PER_TILE)],
                        out_hbm.at[c, pl.ds(r0, ROWS_PER_TILE)])
        if with_cnt:
            pltpu.sync_copy(cnt_s.at[pl.ds(r0, ROWS_PER_TILE)],
                            cnt_hbm.at[c, pl.ds(r0, ROWS_PER_TILE)])

    return pl.kernel(body, out_type=tuple(out_type), mesh=mesh,
                     scratch_types=scratch,
                     compiler_params=pltpu.CompilerParams(
                         use_tc_tiling_on_sc=False))


_edge_agg_cnt = _make_edge_agg(True)
_edge_agg = _make_edge_agg(False)


_BG = B // NW


def _batch_gather_body(tab_hbm, x_hbm, out_hbm, idx_v, rows_v, sem):
    c = lax.axis_index("c")
    s = lax.axis_index("s")
    base = (s * NC + c) * _BG
    pltpu.sync_copy(x_hbm.at[pl.ds(base, _BG)], idx_v)
    pltpu.async_copy(tab_hbm.at[idx_v], rows_v, sem).wait()
    pltpu.sync_copy(rows_v, out_hbm.at[pl.ds(base, _BG)])


_batch_gather = pl.kernel(
    _batch_gather_body,
    out_type=jax.ShapeDtypeStruct((B, EMB), jnp.float32),
    mesh=plsc.VectorSubcoreMesh(core_axis_name="c", subcore_axis_name="s"),
    scratch_types=[
        pltpu.VMEM((_BG,), jnp.int32),
        pltpu.VMEM((_BG, EMB), jnp.float32),
        pltpu.SemaphoreType.DMA,
    ],
    compiler_params=pltpu.CompilerParams(use_tc_tiling_on_sc=False),
)


_RB = 1000


def _pre_body(nx_ref, rat_ref, w1lT_ref, w1rT_ref, wencT_ref,
              p1_ref, xr_ref, renc_ref):
    nx = nx_ref[...]
    p1_ref[...] = jnp.dot(nx, w1lT_ref[...],
                          preferred_element_type=jnp.float32)
    xr_ref[...] = jnp.dot(nx, w1rT_ref[...],
                          preferred_element_type=jnp.float32)
    renc_ref[...] = jnp.dot(rat_ref[...], wencT_ref[...],
                            preferred_element_type=jnp.float32)


def _tc_pre(node_x, rating_mat, w1lT, w1rT, wencT):
    n = node_x.shape[0]
    grid = (n // _RB,)
    return pl.pallas_call(
        _pre_body,
        grid=grid,
        in_specs=[
            pl.BlockSpec((_RB, D_FEAT), lambda i: (i, 0)),
            pl.BlockSpec((_RB, M_ITEMS), lambda i: (i, 0)),
            pl.BlockSpec((D_FEAT, EMB), lambda i: (0, 0)),
            pl.BlockSpec((D_FEAT, EMB), lambda i: (0, 0)),
            pl.BlockSpec((M_ITEMS, EMB), lambda i: (0, 0)),
        ],
        out_specs=[
            pl.BlockSpec((_RB, EMB), lambda i: (i, 0)),
            pl.BlockSpec((_RB, EMB), lambda i: (i, 0)),
            pl.BlockSpec((_RB, EMB), lambda i: (i, 0)),
        ],
        out_shape=[
            jax.ShapeDtypeStruct((n, EMB), jnp.float32),
            jax.ShapeDtypeStruct((n, EMB), jnp.float32),
            jax.ShapeDtypeStruct((n, EMB), jnp.float32),
        ],
    )(node_x, rating_mat, w1lT, w1rT, wencT)


def _mid_body(agg_ref, cnt_ref, xr_ref, mask2_ref, b1l_ref,
              w2lT_ref, w2rT_ref, p2_ref, hr_ref):
    a = agg_ref[0] + agg_ref[1]
    cnt = cnt_ref[0, :, 0:1] + cnt_ref[1, :, 0:1]
    mean = a / jnp.maximum(cnt, 1.0)
    h = jnp.maximum(mean + b1l_ref[...] + xr_ref[...], 0.0) * mask2_ref[...]
    p2_ref[...] = jnp.dot(h, w2lT_ref[...],
                          preferred_element_type=jnp.float32)
    hr_ref[...] = jnp.dot(h, w2rT_ref[...],
                          preferred_element_type=jnp.float32)


def _tc_mid(agg1, cnt, xr, mask2, b1l, w2lT, w2rT):
    n = xr.shape[0]
    grid = (n // _RB,)
    return pl.pallas_call(
        _mid_body,
        grid=grid,
        in_specs=[
            pl.BlockSpec((NC, _RB, EMB), lambda i: (0, i, 0)),
            pl.BlockSpec((NC, _RB, 16), lambda i: (0, i, 0)),
            pl.BlockSpec((_RB, EMB), lambda i: (i, 0)),
            pl.BlockSpec((_RB, EMB), lambda i: (i, 0)),
            pl.BlockSpec((1, EMB), lambda i: (0, 0)),
            pl.BlockSpec((EMB, EMB), lambda i: (0, 0)),
            pl.BlockSpec((EMB, EMB), lambda i: (0, 0)),
        ],
        out_specs=[
            pl.BlockSpec((_RB, EMB), lambda i: (i, 0)),
            pl.BlockSpec((_RB, EMB), lambda i: (i, 0)),
        ],
        out_shape=[
            jax.ShapeDtypeStruct((n, EMB), jnp.float32),
            jax.ShapeDtypeStruct((n, EMB), jnp.float32),
        ],
    )(agg1, cnt, xr, mask2, b1l, w2lT, w2rT)


def _post_body(agg_ref, cnt_ref, hr_ref, renc_ref, bias_ref, s_ref):
    a = agg_ref[0] + agg_ref[1]
    cnt = cnt_ref[0, :, 0:1] + cnt_ref[1, :, 0:1]
    mean = a / jnp.maximum(cnt, 1.0)
    s_ref[...] = mean + hr_ref[...] + renc_ref[...] + bias_ref[...]


def _tc_post(agg2, cnt, hr, renc, bias):
    n = hr.shape[0]
    grid = (n // _RB,)
    return pl.pallas_call(
        _post_body,
        grid=grid,
        in_specs=[
            pl.BlockSpec((NC, _RB, EMB), lambda i: (0, i, 0)),
            pl.BlockSpec((NC, _RB, 16), lambda i: (0, i, 0)),
            pl.BlockSpec((_RB, EMB), lambda i: (i, 0)),
            pl.BlockSpec((_RB, EMB), lambda i: (i, 0)),
            pl.BlockSpec((1, EMB), lambda i: (0, 0)),
        ],
        out_specs=pl.BlockSpec((_RB, EMB), lambda i: (i, 0)),
        out_shape=jax.ShapeDtypeStruct((n, EMB), jnp.float32),
    )(agg2, cnt, hr, renc, bias)


_DB = 512


def _dec_body(sx_ref, wdecT_ref, bdec_ref, out_ref):
    t = jax.nn.sigmoid(sx_ref[...])
    y = jnp.dot(t, wdecT_ref[...], preferred_element_type=jnp.float32)
    out_ref[...] = jax.nn.sigmoid(y + bdec_ref[...])


def _tc_dec(sx, wdecT, bdec):
    grid = (B // _DB,)
    return pl.pallas_call(
        _dec_body,
        grid=grid,
        in_specs=[
            pl.BlockSpec((_DB, EMB), lambda i: (i, 0)),
            pl.BlockSpec((EMB, M_ITEMS), lambda i: (0, 0)),
            pl.BlockSpec((1, M_ITEMS), lambda i: (0, 0)),
        ],
        out_specs=pl.BlockSpec((_DB, M_ITEMS), lambda i: (i, 0)),
        out_shape=jax.ShapeDtypeStruct((B, M_ITEMS), jnp.float32),
    )(sx, wdecT, bdec)


def kernel(x, rating_mat, node_x, edge_index, user_table,
           w1l, b1l, w1r, w2l, b2l, w2r,
           w_enc, b_enc, w_dec, b_dec):
    del user_table
    x = x.astype(jnp.int32)
    src = edge_index[0].astype(jnp.int32)
    dst = edge_index[1].astype(jnp.int32)
    pad = E_PAD - N_EDGES
    pad_dst = N_NODES + jnp.arange(pad, dtype=jnp.int32) % (N_ACC - N_NODES)
    src_p = jnp.concatenate([src, jnp.zeros((pad,), jnp.int32)])
    dst_p = jnp.concatenate([dst, pad_dst])
    src_p = src_p.reshape(NW, NCH, CH)
    dst_p = dst_p.reshape(NW, NCH, CH)
    z64 = jnp.zeros((N_ACC, EMB), jnp.float32)
    z16 = jnp.zeros((N_ACC, 16), jnp.float32)
    ones = jnp.ones((CH, 16), jnp.float32)

    keep = jax.random.bernoulli(jax.random.key(42), 0.5, (N_NODES, EMB))
    mask2 = keep.astype(jnp.float32) * 2.0

    p1, xr, renc = _tc_pre(node_x, rating_mat, _f32(w1l.T), _f32(w1r.T),
                           _f32(w_enc.T))
    agg1, cnt = _edge_agg_cnt(p1, src_p, dst_p, z64, z16, ones)
    p2, hr = _tc_mid(agg1, cnt, xr, mask2, b1l.reshape(1, EMB),
                     _f32(w2l.T), _f32(w2r.T))
    (agg2,) = _edge_agg(p2, src_p, dst_p, z64, z16, ones)
    bias = (b2l + b_enc).reshape(1, EMB)
    s_tab = _tc_post(agg2, cnt, hr, renc, bias)
    sx = _batch_gather(s_tab, x)
    return _tc_dec(sx, _f32(w_dec.T), b_dec.reshape(1, M_ITEMS))
